# Initial kernel scaffold; baseline (speedup 1.0000x reference)
import jax
import jax.numpy as jnp
from jax.experimental import pallas as pl


def kernel(feats, W1, b1, Wh, bh, Wt, bt, Wl1, bl1, Wl2, bl2, gamma, beta, Wc, bc):
    raise NotImplementedError("write your pallas kernel here")


# speedup vs baseline: 24.8726x; 24.8726x over previous
import functools

import jax
import jax.numpy as jnp
from jax import lax
from jax.experimental import pallas as pl
from jax.experimental.pallas import tpu as pltpu

N = 4096
DIM_IN = 384
DIM_H = 64
TOPK = 6
ROW_BLK = 256
NUM_BLKS = N // ROW_BLK
NEG = -1e30


def _fc1_body(feats_ref, w1t_ref, b1_ref, out_ref):
    v = jnp.dot(feats_ref[...], w1t_ref[...],
                preferred_element_type=jnp.float32) + b1_ref[...]
    out_ref[...] = jnp.where(v >= 0, v, 0.01 * v)


def _leaky(v):
    return jnp.where(v >= 0, v, 0.01 * v)


def _main_body(x_ref, wht_ref, bh_ref, wtt_ref, bt_ref,
               wl1t_ref, bl1_ref, wl2t_ref, bl2_ref,
               gamma_ref, beta_ref, wct_ref, bc_ref,
               out_ref, eh_s, et_s, acc_s):
    i = pl.program_id(0)

    @pl.when(i == 0)
    def _prologue():
        x = x_ref[...]
        xm = jnp.mean(x, axis=0, keepdims=True)
        x = (x + xm) * 0.5
        eh_s[...] = jnp.dot(x, wht_ref[...],
                            preferred_element_type=jnp.float32) + bh_ref[...]
        et_s[...] = jnp.dot(x, wtt_ref[...],
                            preferred_element_type=jnp.float32) + bt_ref[...]
        acc_s[...] = jnp.zeros_like(acc_s)

    rows = pl.ds(i * ROW_BLK, ROW_BLK)
    e_h = eh_s[rows, :]
    e_t = et_s[...]
    scale = DIM_H ** (-0.5)
    logits = lax.dot_general(e_h * scale, e_t,
                             (((1,), (1,)), ((), ())),
                             preferred_element_type=jnp.float32)

    iota = lax.broadcasted_iota(jnp.int32, (ROW_BLK, N), 1)
    ws = []
    nbs = []
    for _ in range(TOPK):
        m = jnp.max(logits, axis=1, keepdims=True)
        idx = jnp.min(jnp.where(logits == m, iota, N), axis=1,
                      keepdims=True)
        onehot = (iota == idx)
        ws.append(m)
        nbs.append(jnp.dot(onehot.astype(jnp.float32), e_t,
                           preferred_element_type=jnp.float32))
        logits = jnp.where(onehot, NEG, logits)

    exps = [jnp.exp(w - ws[0]) for w in ws]
    z = functools.reduce(jnp.add, exps)
    ps = [e / z for e in exps]

    kas = []
    for p, nb in zip(ps, nbs):
        eh_r = p * nb + (1.0 - p) * e_h
        gate = jnp.tanh(e_h + eh_r)
        kas.append(jnp.sum(nb, axis=1, keepdims=True) *
                   jnp.sum(gate, axis=1, keepdims=True))
    ka_max = functools.reduce(jnp.maximum, kas)
    qs = [jnp.exp(ka - ka_max) for ka in kas]
    qz = functools.reduce(jnp.add, qs)
    e_nh = functools.reduce(
        jnp.add, [(q / qz) * nb for q, nb in zip(qs, nbs)])

    sum_emb = _leaky(jnp.dot(e_h + e_nh, wl1t_ref[...],
                             preferred_element_type=jnp.float32) + bl1_ref[...])
    bi_emb = _leaky(jnp.dot(e_h * e_nh, wl2t_ref[...],
                            preferred_element_type=jnp.float32) + bl2_ref[...])
    emb = sum_emb + bi_emb
    acc_s[...] += jnp.sum(emb, axis=0, keepdims=True)

    @pl.when(i == NUM_BLKS - 1)
    def _epilogue():
        h = acc_s[...] / N
        mu = jnp.mean(h, axis=1, keepdims=True)
        var = jnp.mean((h - mu) ** 2, axis=1, keepdims=True)
        hn = (h - mu) / jnp.sqrt(var + 1e-5) * gamma_ref[...] + beta_ref[...]
        out_ref[...] = jnp.dot(hn, wct_ref[...],
                               preferred_element_type=jnp.float32) + bc_ref[...]


def kernel(feats, W1, b1, Wh, bh, Wt, bt, Wl1, bl1, Wl2, bl2, gamma, beta, Wc, bc):
    f2 = feats.reshape(N, DIM_IN)

    x_raw = pl.pallas_call(
        _fc1_body,
        grid=(NUM_BLKS,),
        in_specs=[
            pl.BlockSpec((ROW_BLK, DIM_IN), lambda i: (i, 0)),
            pl.BlockSpec((DIM_IN, DIM_H), lambda i: (0, 0)),
            pl.BlockSpec((1, DIM_H), lambda i: (0, 0)),
        ],
        out_specs=pl.BlockSpec((ROW_BLK, DIM_H), lambda i: (i, 0)),
        out_shape=jax.ShapeDtypeStruct((N, DIM_H), jnp.float32),
    )(f2, W1.T, b1.reshape(1, DIM_H))

    full = lambda s: pl.BlockSpec(s, lambda i: tuple(0 for _ in s))
    out = pl.pallas_call(
        _main_body,
        grid=(NUM_BLKS,),
        in_specs=[
            full((N, DIM_H)),
            full((DIM_H, DIM_H)), full((1, DIM_H)),
            full((DIM_H, DIM_H)), full((1, DIM_H)),
            full((DIM_H, DIM_H)), full((1, DIM_H)),
            full((DIM_H, DIM_H)), full((1, DIM_H)),
            full((1, DIM_H)), full((1, DIM_H)),
            full((DIM_H, 2)), full((1, 2)),
        ],
        out_specs=full((1, 2)),
        out_shape=jax.ShapeDtypeStruct((1, 2), jnp.float32),
        scratch_shapes=[
            pltpu.VMEM((N, DIM_H), jnp.float32),
            pltpu.VMEM((N, DIM_H), jnp.float32),
            pltpu.VMEM((1, DIM_H), jnp.float32),
        ],
    )(x_raw, Wh.T, bh.reshape(1, DIM_H), Wt.T, bt.reshape(1, DIM_H),
      Wl1.T, bl1.reshape(1, DIM_H), Wl2.T, bl2.reshape(1, DIM_H),
      gamma.reshape(1, DIM_H), beta.reshape(1, DIM_H),
      Wc.T, bc.reshape(1, 2))
    return out
